# baseline (speedup 1.0000x reference)
import jax
import jax.numpy as jnp
from jax.experimental import pallas as pl

NUM_ATTR = 8
DIM = 128
BATCH = 16384
EPS = 1e-06

_BLK = 16384
_PK = _BLK // 128


def _tc_body(x_ref, a_ref, mu_ref, sig_ref, o_ref):
    inv8 = 1.0 / (jnp.log1p(jnp.exp(sig_ref[...])) + EPS)
    mu8 = mu_ref[...]
    at = a_ref[...]
    mu_pk = jnp.zeros(at.shape, jnp.float32)
    inv_pk = jnp.zeros(at.shape, jnp.float32)
    for a in range(NUM_ATTR):
        m = at == a
        mu_pk = jnp.where(m, mu8[0, a], mu_pk)
        inv_pk = jnp.where(m, inv8[0, a], inv_pk)
    mt = jnp.swapaxes(mu_pk, 0, 1)
    it = jnp.swapaxes(inv_pk, 0, 1)
    for k in range(_PK):
        xk = x_ref[k * 128:(k + 1) * 128, :]
        o_ref[k * 128:(k + 1) * 128, :] = (xk - mt[:, k:k + 1]) * it[:, k:k + 1]


@jax.jit
def kernel(x, attr, mus, sigmas):
    attr_pk = attr.astype(jnp.int32).reshape(BATCH // 128, 128)
    mus2 = mus.reshape(1, NUM_ATTR)
    sig2 = sigmas.reshape(1, NUM_ATTR)
    grid = BATCH // _BLK
    return pl.pallas_call(
        _tc_body,
        grid=(grid,),
        in_specs=[
            pl.BlockSpec((_BLK, DIM), lambda i: (i, 0)),
            pl.BlockSpec((_PK, 128), lambda i: (i, 0)),
            pl.BlockSpec((1, NUM_ATTR), lambda i: (0, 0)),
            pl.BlockSpec((1, NUM_ATTR), lambda i: (0, 0)),
        ],
        out_specs=pl.BlockSpec((_BLK, DIM), lambda i: (i, 0)),
        out_shape=jax.ShapeDtypeStruct((BATCH, DIM), jnp.float32),
    )(x, attr_pk, mus2, sig2)


# speedup vs baseline: 1.2146x; 1.2146x over previous
import jax
import jax.numpy as jnp
from jax.experimental import pallas as pl

NUM_ATTR = 8
DIM = 128
BATCH = 16384
EPS = 1e-06

_BLK = 8192
_PK = _BLK // 128


def _tc_body(x_ref, a_ref, mu_ref, sig_ref, o_ref):
    inv8 = 1.0 / (jnp.log1p(jnp.exp(sig_ref[...])) + EPS)
    mu8 = mu_ref[...]
    at = a_ref[...]
    mu_pk = jnp.zeros(at.shape, jnp.float32)
    inv_pk = jnp.zeros(at.shape, jnp.float32)
    for a in range(NUM_ATTR):
        m = at == a
        mu_pk = jnp.where(m, mu8[0, a], mu_pk)
        inv_pk = jnp.where(m, inv8[0, a], inv_pk)
    mt = jnp.swapaxes(mu_pk, 0, 1)
    it = jnp.swapaxes(inv_pk, 0, 1)
    for k in range(_PK):
        xk = x_ref[k * 128:(k + 1) * 128, :]
        o_ref[k * 128:(k + 1) * 128, :] = (xk - mt[:, k:k + 1]) * it[:, k:k + 1]


@jax.jit
def kernel(x, attr, mus, sigmas):
    attr_pk = attr.astype(jnp.int32).reshape(BATCH // 128, 128)
    mus2 = mus.reshape(1, NUM_ATTR)
    sig2 = sigmas.reshape(1, NUM_ATTR)
    grid = BATCH // _BLK
    return pl.pallas_call(
        _tc_body,
        grid=(grid,),
        in_specs=[
            pl.BlockSpec((_BLK, DIM), lambda i: (i, 0)),
            pl.BlockSpec((_PK, 128), lambda i: (i, 0)),
            pl.BlockSpec((1, NUM_ATTR), lambda i: (0, 0)),
            pl.BlockSpec((1, NUM_ATTR), lambda i: (0, 0)),
        ],
        out_specs=pl.BlockSpec((_BLK, DIM), lambda i: (i, 0)),
        out_shape=jax.ShapeDtypeStruct((BATCH, DIM), jnp.float32),
    )(x, attr_pk, mus2, sig2)
